# baseline (speedup 1.0000x reference)
import functools

import jax
import jax.numpy as jnp
from jax import lax
from jax.experimental import pallas as pl
from jax.experimental.pallas import tpu as pltpu
from jax.experimental.pallas import tpu_sc as plsc

_VOCAB = 100000
_EMBED = 64
_CTX = 4
_HID = 256
_BATCH = 1024

_NC = 2
_NS = 16
_NW = _NC * _NS
_NIDX = _BATCH * _CTX
_B_PER_W = _NIDX // _NW

_BN = 4864


@functools.partial(
    pl.kernel,
    out_type=jax.ShapeDtypeStruct((_NIDX, _EMBED), jnp.bfloat16),
    mesh=plsc.VectorSubcoreMesh(core_axis_name="c", subcore_axis_name="s"),
    scratch_types=[
        pltpu.VMEM((_B_PER_W,), jnp.int32),
        pltpu.VMEM((_B_PER_W, _EMBED), jnp.bfloat16),
        pltpu.SemaphoreType.DMA,
    ],
    compiler_params=pltpu.CompilerParams(use_tc_tiling_on_sc=False),
)
def _sc_gather(table_hbm, idx_hbm, out_hbm, idx_v, rows_v, sem):
    wid = lax.axis_index("s") * _NC + lax.axis_index("c")
    base = wid * _B_PER_W
    pltpu.sync_copy(idx_hbm.at[pl.ds(base, _B_PER_W)], idx_v)
    pltpu.async_copy(table_hbm.at[idx_v], rows_v, sem).wait()
    pltpu.sync_copy(rows_v, out_hbm.at[pl.ds(base, _B_PER_W)])


def _mlp_body(xt_ref, w1t_ref, b1_ref, w2t_ref, b2_ref, out_ref, ht_ref):
    @pl.when(pl.program_id(0) == 0)
    def _():
        ht_ref[...] = jnp.maximum(w1t_ref[...] @ xt_ref[...] + b1_ref[...], 0.0)

    out_ref[...] = w2t_ref[...] @ ht_ref[...] + jnp.transpose(b2_ref[...])


def kernel(inputs, table, W1, b1, W2, b2):
    idx = inputs.reshape(-1).astype(jnp.int32)
    emb = _sc_gather(table.astype(jnp.bfloat16), idx)
    xt = jnp.transpose(emb.reshape(_BATCH, _CTX * _EMBED)).astype(jnp.float32)

    grid = pl.cdiv(_VOCAB, _BN)
    in_dim = _CTX * _EMBED
    logits_t = pl.pallas_call(
        _mlp_body,
        grid=(grid,),
        in_specs=[
            pl.BlockSpec((in_dim, _BATCH), lambda i: (0, 0)),
            pl.BlockSpec((_HID, in_dim), lambda i: (0, 0)),
            pl.BlockSpec((_HID, 1), lambda i: (0, 0)),
            pl.BlockSpec((_BN, _HID), lambda i: (i, 0)),
            pl.BlockSpec((1, _BN), lambda i: (0, i)),
        ],
        out_specs=pl.BlockSpec((_BN, _BATCH), lambda i: (i, 0)),
        out_shape=jax.ShapeDtypeStruct((_VOCAB, _BATCH), jnp.float32),
        scratch_shapes=[pltpu.VMEM((_HID, _BATCH), jnp.float32)],
    )(xt, jnp.transpose(W1), b1.reshape(_HID, 1), jnp.transpose(W2),
      b2.reshape(1, _VOCAB))
    return jnp.transpose(logits_t)


# speedup vs baseline: 1.1078x; 1.1078x over previous
import functools

import jax
import jax.numpy as jnp
from jax import lax
from jax.experimental import pallas as pl
from jax.experimental.pallas import tpu as pltpu
from jax.experimental.pallas import tpu_sc as plsc

_VOCAB = 100000
_EMBED = 64
_CTX = 4
_HID = 256
_BATCH = 1024

_NC = 2
_NS = 16
_NW = _NC * _NS
_NIDX = _BATCH * _CTX
_B_PER_W = _NIDX // _NW

_BN = 4864


@functools.partial(
    pl.kernel,
    out_type=jax.ShapeDtypeStruct((_NIDX, _EMBED), jnp.float32),
    mesh=plsc.VectorSubcoreMesh(core_axis_name="c", subcore_axis_name="s"),
    scratch_types=[
        pltpu.VMEM((_B_PER_W,), jnp.int32),
        pltpu.VMEM((_B_PER_W, _EMBED), jnp.float32),
        pltpu.SemaphoreType.DMA,
    ],
    compiler_params=pltpu.CompilerParams(use_tc_tiling_on_sc=False),
)
def _sc_gather(table_hbm, idx_hbm, out_hbm, idx_v, rows_v, sem):
    wid = lax.axis_index("s") * _NC + lax.axis_index("c")
    base = wid * _B_PER_W
    pltpu.sync_copy(idx_hbm.at[pl.ds(base, _B_PER_W)], idx_v)
    pltpu.async_copy(table_hbm.at[idx_v], rows_v, sem).wait()
    pltpu.sync_copy(rows_v, out_hbm.at[pl.ds(base, _B_PER_W)])


_MBN = 2048
_NSTEP = _VOCAB // _MBN
_TAIL = _VOCAB - _NSTEP * _MBN
_NBUF = 4


def _mlp_body(xt_ref, w1t_ref, b1_ref, b2_ref, w2t_hbm, out_hbm, ht_ref,
              w2_bufs, out_bufs, in_sems, out_sems):
    ht_ref[...] = jnp.maximum(w1t_ref[...] @ xt_ref[...] + b1_ref[...], 0.0)

    def in_cp(step, slot):
        return pltpu.make_async_copy(
            w2t_hbm.at[pl.ds(step * _MBN, _MBN), :], w2_bufs.at[slot],
            in_sems.at[slot])

    def out_cp(step, slot):
        return pltpu.make_async_copy(
            out_bufs.at[slot], out_hbm.at[pl.ds(step * _MBN, _MBN), :],
            out_sems.at[slot])

    for p in range(_NBUF):
        in_cp(p, p).start()
    for i in range(_NSTEP):
        slot = i % _NBUF
        in_cp(i, slot).wait()
        if i >= _NBUF:
            out_cp(i - _NBUF, slot).wait()
        out_bufs[slot] = (w2_bufs[slot] @ ht_ref[...]
                          + jnp.transpose(b2_ref[:, pl.ds(i * _MBN, _MBN)]))
        out_cp(i, slot).start()
        nxt = i + _NBUF
        if nxt < _NSTEP:
            in_cp(nxt, slot).start()
    out_cp(_NSTEP - _NBUF, 0).wait()
    tin = pltpu.make_async_copy(
        w2t_hbm.at[pl.ds(_NSTEP * _MBN, _TAIL), :],
        w2_bufs.at[0, pl.ds(0, _TAIL), :], in_sems.at[0])
    tin.start()
    tin.wait()
    out_bufs[0, pl.ds(0, _TAIL), :] = (
        w2_bufs[0, pl.ds(0, _TAIL), :] @ ht_ref[...]
        + jnp.transpose(b2_ref[:, pl.ds(_NSTEP * _MBN, _TAIL)]))
    tout = pltpu.make_async_copy(
        out_bufs.at[0, pl.ds(0, _TAIL), :],
        out_hbm.at[pl.ds(_NSTEP * _MBN, _TAIL), :], out_sems.at[0])
    tout.start()
    for i in range(_NSTEP - _NBUF + 1, _NSTEP):
        out_cp(i, i % _NBUF).wait()
    tout.wait()


def kernel(inputs, table, W1, b1, W2, b2):
    idx = inputs.reshape(-1).astype(jnp.int32)
    emb = _sc_gather(table, idx)
    xt = jnp.transpose(emb.reshape(_BATCH, _CTX * _EMBED))

    in_dim = _CTX * _EMBED
    logits_t = pl.pallas_call(
        _mlp_body,
        in_specs=[
            pl.BlockSpec(memory_space=pltpu.VMEM),
            pl.BlockSpec(memory_space=pltpu.VMEM),
            pl.BlockSpec(memory_space=pltpu.VMEM),
            pl.BlockSpec(memory_space=pltpu.VMEM),
            pl.BlockSpec(memory_space=pltpu.HBM),
        ],
        out_specs=pl.BlockSpec(memory_space=pltpu.HBM),
        out_shape=jax.ShapeDtypeStruct((_VOCAB, _BATCH), jnp.float32),
        scratch_shapes=[
            pltpu.VMEM((_HID, _BATCH), jnp.float32),
            pltpu.VMEM((_NBUF, _MBN, _HID), jnp.float32),
            pltpu.VMEM((_NBUF, _MBN, _BATCH), jnp.float32),
            pltpu.SemaphoreType.DMA((_NBUF,)),
            pltpu.SemaphoreType.DMA((_NBUF,)),
        ],
    )(xt, jnp.transpose(W1), b1.reshape(_HID, 1), b2.reshape(1, _VOCAB),
      jnp.transpose(W2))
    return jnp.transpose(logits_t)


# speedup vs baseline: 1.1333x; 1.0230x over previous
import functools

import jax
import jax.numpy as jnp
from jax import lax
from jax.experimental import pallas as pl
from jax.experimental.pallas import tpu as pltpu
from jax.experimental.pallas import tpu_sc as plsc

_VOCAB = 100000
_EMBED = 64
_CTX = 4
_HID = 256
_BATCH = 1024

_NC = 2
_NS = 16
_NW = _NC * _NS
_NIDX = _BATCH * _CTX
_B_PER_W = _NIDX // _NW

_BN = 4864


@functools.partial(
    pl.kernel,
    out_type=jax.ShapeDtypeStruct((_NIDX, _EMBED), jnp.float32),
    mesh=plsc.VectorSubcoreMesh(core_axis_name="c", subcore_axis_name="s"),
    scratch_types=[
        pltpu.VMEM((_B_PER_W,), jnp.int32),
        pltpu.VMEM((_B_PER_W, _EMBED), jnp.float32),
        pltpu.SemaphoreType.DMA,
    ],
    compiler_params=pltpu.CompilerParams(use_tc_tiling_on_sc=False),
)
def _sc_gather(table_hbm, idx_hbm, out_hbm, idx_v, rows_v, sem):
    wid = lax.axis_index("s") * _NC + lax.axis_index("c")
    base = wid * _B_PER_W
    pltpu.sync_copy(idx_hbm.at[pl.ds(base, _B_PER_W)], idx_v)
    pltpu.async_copy(table_hbm.at[idx_v], rows_v, sem).wait()
    pltpu.sync_copy(rows_v, out_hbm.at[pl.ds(base, _B_PER_W)])


def _mlp_body(xt_ref, w1t_ref, b1_ref, w2t_ref, b2_ref, out_ref, ht_ref):
    @pl.when(pl.program_id(0) == 0)
    def _():
        ht_ref[...] = jnp.maximum(w1t_ref[...] @ xt_ref[...] + b1_ref[...], 0.0)

    out_ref[...] = w2t_ref[...] @ ht_ref[...] + jnp.transpose(b2_ref[...])


def kernel(inputs, table, W1, b1, W2, b2):
    idx = inputs.reshape(-1).astype(jnp.int32)
    emb = _sc_gather(table, idx)
    xt = jnp.transpose(emb.reshape(_BATCH, _CTX * _EMBED))

    grid = pl.cdiv(_VOCAB, _BN)
    in_dim = _CTX * _EMBED
    logits_t = pl.pallas_call(
        _mlp_body,
        grid=(grid,),
        in_specs=[
            pl.BlockSpec((in_dim, _BATCH), lambda i: (0, 0)),
            pl.BlockSpec((_HID, in_dim), lambda i: (0, 0)),
            pl.BlockSpec((_HID, 1), lambda i: (0, 0)),
            pl.BlockSpec((_BN, _HID), lambda i: (i, 0)),
            pl.BlockSpec((1, _BN), lambda i: (0, i)),
        ],
        out_specs=pl.BlockSpec((_BN, _BATCH), lambda i: (i, 0)),
        out_shape=jax.ShapeDtypeStruct((_VOCAB, _BATCH), jnp.float32),
        scratch_shapes=[pltpu.VMEM((_HID, _BATCH), jnp.float32)],
    )(xt, jnp.transpose(W1), b1.reshape(_HID, 1), jnp.transpose(W2),
      b2.reshape(1, _VOCAB))
    return jnp.transpose(logits_t)
